# Initial kernel scaffold; baseline (speedup 1.0000x reference)
import jax
import jax.numpy as jnp
from jax.experimental import pallas as pl


def kernel(inputs, table):
    raise NotImplementedError("write your pallas kernel here")


# speedup vs baseline: 2.9688x; 2.9688x over previous
import functools

import jax
import jax.numpy as jnp
from jax import lax
from jax.experimental import pallas as pl
from jax.experimental.pallas import tpu as pltpu
from jax.experimental.pallas import tpu_sc as plsc


def _gather_kernel(n_workers, n_chunks, chunk, d):
  mesh = plsc.VectorSubcoreMesh(core_axis_name="c", subcore_axis_name="s")
  n_per_w = n_chunks * chunk

  @functools.partial(
      pl.kernel,
      mesh=mesh,
      out_type=jax.ShapeDtypeStruct((n_workers * n_per_w, d), jnp.float32),
      scratch_types=[
          pltpu.VMEM((n_chunks, chunk), jnp.int32),
          pltpu.VMEM((2, chunk, d), jnp.float32),
          pltpu.SemaphoreType.DMA,
      ],
  )
  def k(idx_hbm, table_hbm, out_hbm, idx_v, rows_v, gsem):
    wid = lax.axis_index("s") * 2 + lax.axis_index("c")
    base = wid * n_per_w
    pltpu.sync_copy(idx_hbm.at[wid], idx_v)

    def body(j, _):
      pltpu.async_copy(table_hbm.at[idx_v.at[j]], rows_v.at[0], gsem).wait()
      pltpu.sync_copy(rows_v.at[0], out_hbm.at[pl.ds(base + j * chunk, chunk)])
      return 0

    lax.fori_loop(0, n_chunks, body, 0)

  return k


def kernel(inputs, table):
  b, s = inputs.shape
  v, d = table.shape
  n = b * s
  n_workers = 32
  chunk = 128
  n_per_w = n // n_workers
  n_chunks = n_per_w // chunk
  idx = inputs.astype(jnp.int32).reshape(n_workers, n_chunks, chunk)
  out = _gather_kernel(n_workers, n_chunks, chunk, d)(idx, table)
  return out.reshape(b, s, d)


# speedup vs baseline: 3.3387x; 1.1246x over previous
import functools

import jax
import jax.numpy as jnp
from jax import lax
from jax.experimental import pallas as pl
from jax.experimental.pallas import tpu as pltpu
from jax.experimental.pallas import tpu_sc as plsc


def _gather_kernel(n_workers, n_chunks, chunk, d, group):
  mesh = plsc.VectorSubcoreMesh(core_axis_name="c", subcore_axis_name="s")
  n_per_w = n_chunks * chunk
  n_steps = n_chunks // group

  @functools.partial(
      pl.kernel,
      mesh=mesh,
      out_type=jax.ShapeDtypeStruct((n_workers * n_per_w, d), jnp.float32),
      scratch_types=[
          pltpu.VMEM((n_chunks, chunk), jnp.int32),
          pltpu.VMEM((2 * group, chunk, d), jnp.float32),
          pltpu.SemaphoreType.DMA,
          pltpu.SemaphoreType.DMA,
          pltpu.SemaphoreType.DMA,
          pltpu.SemaphoreType.DMA,
      ],
  )
  def k(idx_hbm, table_hbm, out_hbm, idx_v, rows_v, g0, g1, s0, s1):
    wid = lax.axis_index("s") * 2 + lax.axis_index("c")
    base = wid * n_per_w
    pltpu.sync_copy(idx_hbm.at[wid], idx_v)
    gsems = (g0, g1)
    ssems = (s0, s1)

    def gather(j, b, sem):
      return pltpu.make_async_copy(table_hbm.at[idx_v.at[j]], rows_v.at[b],
                                   sem)

    def scatter(j, b, sem):
      return pltpu.make_async_copy(
          rows_v.at[b], out_hbm.at[pl.ds(base + j * chunk, chunk)], sem)

    def step(t, g, first, last):
      og = 1 - g
      if not first:
        for i in range(group):
          scatter((t - 1) * group + i, og * group + i, ssems[og]).wait()
      if not last:
        for i in range(group):
          gather((t + 1) * group + i, og * group + i, gsems[og]).start()
      for i in range(group):
        gather(t * group + i, g * group + i, gsems[g]).wait()
      for i in range(group):
        scatter(t * group + i, g * group + i, ssems[g]).start()

    for i in range(group):
      gather(i, i, gsems[0]).start()
    step(0, 0, first=True, last=False)
    def body(u, _):
      step(2 * u + 1, 1, first=False, last=False)
      step(2 * u + 2, 0, first=False, last=False)
      return 0

    lax.fori_loop(0, (n_steps - 3) // 2, body, 0)
    step(n_steps - 2, (n_steps - 2) % 2, first=False, last=False)
    step(n_steps - 1, (n_steps - 1) % 2, first=False, last=True)
    for i in range(group):
      g = (n_steps - 1) % 2
      scatter((n_steps - 1) * group + i, g * group + i, ssems[g]).wait()

  return k


def kernel(inputs, table):
  b, s = inputs.shape
  v, d = table.shape
  n = b * s
  n_workers = 32
  chunk = 128
  n_per_w = n // n_workers
  n_chunks = n_per_w // chunk
  idx = inputs.astype(jnp.int32).reshape(n_workers, n_chunks, chunk)
  out = _gather_kernel(n_workers, n_chunks, chunk, d, 2)(idx, table)
  return out.reshape(b, s, d)


# speedup vs baseline: 3.3415x; 1.0008x over previous
import functools

import jax
import jax.numpy as jnp
from jax import lax
from jax.experimental import pallas as pl
from jax.experimental.pallas import tpu as pltpu
from jax.experimental.pallas import tpu_sc as plsc


def _gather_kernel(n_workers, n_chunks, chunk, d, group):
  mesh = plsc.VectorSubcoreMesh(core_axis_name="c", subcore_axis_name="s")
  n_steps = n_chunks // group
  assert n_steps * group == n_chunks
  assert n_steps >= 3 and n_steps % 2 == 1
  nbuf = 2 * group

  @functools.partial(
      pl.kernel,
      mesh=mesh,
      out_type=jax.ShapeDtypeStruct(
          (n_workers * n_chunks, chunk, d), jnp.float32),
      scratch_types=[
          pltpu.VMEM((n_chunks, chunk), jnp.int32),
          pltpu.VMEM((nbuf, chunk, d), jnp.float32),
          pltpu.SemaphoreType.DMA,
          pltpu.SemaphoreType.DMA,
          pltpu.SemaphoreType.DMA,
          pltpu.SemaphoreType.DMA,
      ],
  )
  def k(idx_hbm, table_hbm, out_hbm, idx_v, rows_v, g0, g1, s0, s1):
    wid = lax.axis_index("s") * 2 + lax.axis_index("c")
    base = wid * n_chunks
    pltpu.sync_copy(idx_hbm.at[wid], idx_v)
    gsems = (g0, g1)
    ssems = (s0, s1)

    def gathers(t, g):
      return [
          pltpu.make_async_copy(table_hbm.at[idx_v.at[t * group + i]],
                                rows_v.at[g * group + i], gsems[g])
          for i in range(group)
      ]

    def scatter(t, g):
      return pltpu.make_async_copy(
          rows_v.at[pl.ds(g * group, group)],
          out_hbm.at[pl.ds(base + t * group, group)], ssems[g])

    def step(t, g, first, last):
      og = 1 - g
      if not first:
        scatter(t - 1, og).wait()
      if not last:
        for c in gathers(t + 1, og):
          c.start()
      for c in gathers(t, g):
        c.wait()
      scatter(t, g).start()

    for c in gathers(0, 0):
      c.start()
    step(0, 0, first=True, last=False)

    def body(u, _):
      step(2 * u + 1, 1, first=False, last=False)
      step(2 * u + 2, 0, first=False, last=False)
      return 0

    lax.fori_loop(0, (n_steps - 3) // 2, body, 0)
    step(n_steps - 2, (n_steps - 2) % 2, first=False, last=False)
    step(n_steps - 1, (n_steps - 1) % 2, first=False, last=True)
    scatter(n_steps - 1, (n_steps - 1) % 2).wait()

  return k


def kernel(inputs, table):
  b, s = inputs.shape
  v, d = table.shape
  n = b * s
  n_workers = 32
  chunk = 128
  n_per_w = n // n_workers
  n_chunks = n_per_w // chunk
  idx = inputs.astype(jnp.int32).reshape(n_workers, n_chunks, chunk)
  out = _gather_kernel(n_workers, n_chunks, chunk, d, 2)(idx, table)
  return out.reshape(b, s, d)
